# Initial kernel scaffold; baseline (speedup 1.0000x reference)
import jax
import jax.numpy as jnp
from jax.experimental import pallas as pl


def kernel(scores, masks):
    raise NotImplementedError("write your pallas kernel here")


# speedup vs baseline: 47.7414x; 47.7414x over previous
import jax
import jax.numpy as jnp
from jax.experimental import pallas as pl
from jax.experimental.pallas import tpu as pltpu

L, B, N = 4, 1024, 8192
BS = 512
BR = 64
ITERS = 14


def _body(scores_ref, masks_ref, out_ref):
    i = pl.program_id(0)
    x = scores_ref[...]
    m = masks_ref[...]
    kf = jnp.ceil(jnp.sum(m, axis=-1) * 0.1)

    lo = jnp.zeros((L, BR), jnp.float32)
    hi = jnp.ones((L, BR), jnp.float32)
    for _ in range(ITERS):
        mid = 0.5 * (lo + hi)
        cnt = jnp.sum((x >= mid[:, :, None]).astype(jnp.float32), axis=-1)
        pred = cnt >= kf
        lo = jnp.where(pred, mid, lo)
        hi = jnp.where(pred, hi, mid)

    t = lo
    ge = (x >= t[:, :, None]).astype(jnp.float32)
    c = jnp.sum(ge, axis=-1)
    s = jnp.sum(x * ge, axis=-1)
    topk = s - (c - kf) * t
    inp = jnp.mean(topk / kf, axis=0, keepdims=True)
    inp = jnp.minimum(inp, 1.0 - 1e-7)

    b_idx = i * BR + jax.lax.broadcasted_iota(jnp.int32, (1, BR), 1)
    target = (b_idx >= BS).astype(jnp.float32)
    log_p = jnp.maximum(jnp.log(inp), -100.0)
    log_1mp = jnp.maximum(jnp.log(1.0 - inp), -100.0)
    partial = -jnp.sum(target * log_p + (1.0 - target) * log_1mp) / B

    @pl.when(i == 0)
    def _init():
        out_ref[0, 0] = 0.0

    out_ref[0, 0] += partial


def _build(interpret=False):
    call = pl.pallas_call(
        _body,
        grid=(B // BR,),
        in_specs=[
            pl.BlockSpec((L, BR, N), lambda i: (0, i, 0)),
            pl.BlockSpec((L, BR, N), lambda i: (0, i, 0)),
        ],
        out_specs=pl.BlockSpec(memory_space=pltpu.SMEM),
        out_shape=jax.ShapeDtypeStruct((1, 1), jnp.float32),
        interpret=interpret,
    )

    def kernel_fn(scores, masks):
        out = call(scores, masks)
        return out[0, 0]

    return kernel_fn


kernel = _build()


# speedup vs baseline: 54.1303x; 1.1338x over previous
import jax
import jax.numpy as jnp
from jax.experimental import pallas as pl
from jax.experimental.pallas import tpu as pltpu

L, B, N = 4, 1024, 8192
BS = 512
BR = 64
ITERS = 12


def _body(scores_ref, masks_ref, out_ref):
    i = pl.program_id(0)
    x = scores_ref[...]
    m = masks_ref[...]
    kf = jnp.ceil(jnp.sum(m, axis=-1) * 0.1)

    lo = jnp.zeros((L, BR), jnp.float32)
    hi = jnp.ones((L, BR), jnp.float32)
    for _ in range(ITERS):
        mid = 0.5 * (lo + hi)
        cnt = jnp.sum((x >= mid[:, :, None]).astype(jnp.float32), axis=-1)
        pred = cnt >= kf
        lo = jnp.where(pred, mid, lo)
        hi = jnp.where(pred, hi, mid)

    t = lo
    ge = (x >= t[:, :, None]).astype(jnp.float32)
    c = jnp.sum(ge, axis=-1)
    s = jnp.sum(x * ge, axis=-1)
    topk = s - (c - kf) * t
    inp = jnp.mean(topk / kf, axis=0, keepdims=True)
    inp = jnp.minimum(inp, 1.0 - 1e-7)

    b_idx = i * BR + jax.lax.broadcasted_iota(jnp.int32, (1, BR), 1)
    target = (b_idx >= BS).astype(jnp.float32)
    log_p = jnp.maximum(jnp.log(inp), -100.0)
    log_1mp = jnp.maximum(jnp.log(1.0 - inp), -100.0)
    partial = -jnp.sum(target * log_p + (1.0 - target) * log_1mp) / B

    @pl.when(i == 0)
    def _init():
        out_ref[0, 0] = 0.0

    out_ref[0, 0] += partial


def _build(interpret=False):
    call = pl.pallas_call(
        _body,
        grid=(B // BR,),
        in_specs=[
            pl.BlockSpec((L, BR, N), lambda i: (0, i, 0)),
            pl.BlockSpec((L, BR, N), lambda i: (0, i, 0)),
        ],
        out_specs=pl.BlockSpec(memory_space=pltpu.SMEM),
        out_shape=jax.ShapeDtypeStruct((1, 1), jnp.float32),
        interpret=interpret,
    )

    def kernel_fn(scores, masks):
        out = call(scores, masks)
        return out[0, 0]

    return kernel_fn


kernel = _build()
